# baseline (speedup 1.0000x reference)
import functools

import jax
import jax.numpy as jnp
from jax.experimental import pallas as pl
from jax.experimental.pallas import tpu as pltpu

_LANE = 128
_BLOCK_BYTES_MAX = 13 * 1024 * 1024


def _ceil_to(x, m):
    return (x + m - 1) // m * m


def _se_kernel(x_ref, w1_ref, w2_ref, o_ref, gate_ref, *, inv_hw, rsub):
    j = pl.program_id(1)

    @pl.when(j == 0)
    def _():
        x = x_ref[0]
        pooled = jnp.sum(x, axis=-1, keepdims=True,
                         dtype=jnp.float32) * inv_hw
        s = jnp.dot(w1_ref[...], pooled,
                    preferred_element_type=jnp.float32)
        s = s * jax.nn.sigmoid(s)
        u = jnp.dot(w2_ref[...], s,
                    preferred_element_type=jnp.float32)
        gate_ref[...] = jax.nn.sigmoid(u)

    rows = pl.ds(j * rsub, rsub)
    gate = gate_ref[rows, :].astype(o_ref.dtype)
    o_ref[0] = x_ref[0, rows, :] * gate


def _block_diag(w, b):
    if b == 1:
        return w
    o, i = w.shape
    eye = jnp.eye(b, dtype=w.dtype)
    full = eye[:, :, None, None] * w[None, None, :, :]
    return full.transpose(0, 2, 1, 3).reshape(b * o, b * i)


def kernel(x_nchw, w_squeeze, w_unsqueeze):
    N, C, H, W = x_nchw.shape
    if w_squeeze.ndim == 4:
        w_squeeze = w_squeeze.reshape(w_squeeze.shape[0], w_squeeze.shape[1])
    if w_unsqueeze.ndim == 4:
        w_unsqueeze = w_unsqueeze.reshape(w_unsqueeze.shape[0],
                                          w_unsqueeze.shape[1])
    S = w_squeeze.shape[0]
    HW = H * W
    HWp = _ceil_to(HW, _LANE)
    dtype = x_nchw.dtype

    blk_one = C * HWp * dtype.itemsize
    B = 1
    for cand in (8, 4, 2):
        if N % cand == 0 and cand * blk_one <= _BLOCK_BYTES_MAX:
            B = cand
            break
    nsub = 2 if (B % 2 == 0 and (B * C // 2) % 8 == 0) else 1
    rsub = B * C // nsub

    x_flat = x_nchw.reshape(N, C, HW)
    if HWp != HW:
        x_flat = jnp.pad(x_flat, ((0, 0), (0, 0), (0, HWp - HW)))
    xb = x_flat.reshape(N // B, B * C, HWp)

    w1 = _block_diag(w_squeeze.astype(jnp.float32), B)
    w2 = _block_diag(w_unsqueeze.astype(jnp.float32), B)

    blk_bytes = B * blk_one
    vmem_limit = int(min(60 << 20, (2 + 2 // nsub) * blk_bytes + (4 << 20)))

    out = pl.pallas_call(
        functools.partial(_se_kernel, inv_hw=1.0 / HW, rsub=rsub),
        out_shape=jax.ShapeDtypeStruct((N // B, B * C, HWp), dtype),
        grid=(N // B, nsub),
        in_specs=[
            pl.BlockSpec((1, B * C, HWp), lambda n, j: (n, 0, 0)),
            pl.BlockSpec((B * S, B * C), lambda n, j: (0, 0)),
            pl.BlockSpec((B * C, B * S), lambda n, j: (0, 0)),
        ],
        out_specs=pl.BlockSpec((1, rsub, HWp), lambda n, j: (n, j, 0)),
        scratch_shapes=[pltpu.VMEM((B * C, 1), jnp.float32)],
        compiler_params=pltpu.CompilerParams(
            dimension_semantics=("parallel", "arbitrary"),
            vmem_limit_bytes=vmem_limit,
        ),
    )(xb, w1, w2)

    out = out.reshape(N, C, HWp)
    if HWp != HW:
        out = out[:, :, :HW]
    return out.reshape(N, C, H, W)


# speedup vs baseline: 1.0641x; 1.0641x over previous
import functools

import jax
import jax.numpy as jnp
from jax.experimental import pallas as pl
from jax.experimental.pallas import tpu as pltpu

_LANE = 128
_BLOCK_BYTES_MAX = 13 * 1024 * 1024


def _ceil_to(x, m):
    return (x + m - 1) // m * m


def _se_kernel(x_ref, w1_ref, w2_ref, o_ref, *, inv_hw):
    x = x_ref[0]
    pooled = jnp.sum(x, axis=-1, keepdims=True,
                     dtype=jnp.float32) * inv_hw
    s = jnp.dot(w1_ref[...], pooled,
                preferred_element_type=jnp.float32)
    s = s * jax.nn.sigmoid(s)
    u = jnp.dot(w2_ref[...], s,
                preferred_element_type=jnp.float32)
    gate = jax.nn.sigmoid(u).astype(x.dtype)
    o_ref[0] = x * gate


def _block_diag(w, b):
    if b == 1:
        return w
    o, i = w.shape
    eye = jnp.eye(b, dtype=w.dtype)
    full = eye[:, :, None, None] * w[None, None, :, :]
    return full.transpose(0, 2, 1, 3).reshape(b * o, b * i)


def kernel(x_nchw, w_squeeze, w_unsqueeze):
    N, C, H, W = x_nchw.shape
    if w_squeeze.ndim == 4:
        w_squeeze = w_squeeze.reshape(w_squeeze.shape[0], w_squeeze.shape[1])
    if w_unsqueeze.ndim == 4:
        w_unsqueeze = w_unsqueeze.reshape(w_unsqueeze.shape[0],
                                          w_unsqueeze.shape[1])
    S = w_squeeze.shape[0]
    HW = H * W
    HWp = _ceil_to(HW, _LANE)
    dtype = x_nchw.dtype

    blk_one = C * HWp * dtype.itemsize
    B = 1
    for cand in (8, 4, 2):
        if N % cand == 0 and cand * blk_one <= _BLOCK_BYTES_MAX:
            B = cand
            break

    x_flat = x_nchw.reshape(N, C, HW)
    if HWp != HW:
        x_flat = jnp.pad(x_flat, ((0, 0), (0, 0), (0, HWp - HW)))
    xb = x_flat.reshape(N // B, B * C, HWp)

    w1 = _block_diag(w_squeeze.astype(jnp.float32), B)
    w2 = _block_diag(w_unsqueeze.astype(jnp.float32), B)

    blk_bytes = B * blk_one
    vmem_limit = int(min(60 << 20, 4 * blk_bytes + (4 << 20)))

    out = pl.pallas_call(
        functools.partial(_se_kernel, inv_hw=1.0 / HW),
        out_shape=jax.ShapeDtypeStruct((N // B, B * C, HWp), dtype),
        grid=(N // B,),
        in_specs=[
            pl.BlockSpec((1, B * C, HWp), lambda n: (n, 0, 0)),
            pl.BlockSpec((B * S, B * C), lambda n: (0, 0)),
            pl.BlockSpec((B * C, B * S), lambda n: (0, 0)),
        ],
        out_specs=pl.BlockSpec((1, B * C, HWp), lambda n: (n, 0, 0)),
        compiler_params=pltpu.CompilerParams(
            dimension_semantics=("parallel",),
            vmem_limit_bytes=vmem_limit,
        ),
    )(xb, w1, w2)

    out = out.reshape(N, C, HWp)
    if HWp != HW:
        out = out[:, :, :HW]
    return out.reshape(N, C, H, W)


# speedup vs baseline: 1.0675x; 1.0032x over previous
import functools

import jax
import jax.numpy as jnp
from jax.experimental import pallas as pl
from jax.experimental.pallas import tpu as pltpu

_LANE = 128


def _ceil_to(x, m):
    return (x + m - 1) // m * m


def _se_manual_kernel(x_hbm, w1_ref, w2_ref, o_hbm, in_buf, out_buf,
                      in_sems, out_sems, *, n_chunks, rows, d_in, d_out,
                      inv_hw):
    def in_copy(j):
        return pltpu.make_async_copy(
            x_hbm.at[pl.ds(j * rows, rows), :], in_buf.at[j % d_in],
            in_sems.at[j % d_in])

    def out_copy(j):
        return pltpu.make_async_copy(
            out_buf.at[j % d_out], o_hbm.at[pl.ds(j * rows, rows), :],
            out_sems.at[j % d_out])

    for k in range(min(d_in, n_chunks)):
        in_copy(k).start()
    for j in range(n_chunks):
        in_copy(j).wait()
        x = in_buf[j % d_in]
        pooled = jnp.sum(x, axis=-1, keepdims=True,
                         dtype=jnp.float32) * inv_hw
        s = jnp.dot(w1_ref[...], pooled,
                    preferred_element_type=jnp.float32)
        s = s * jax.nn.sigmoid(s)
        u = jnp.dot(w2_ref[...], s,
                    preferred_element_type=jnp.float32)
        gate = jax.nn.sigmoid(u).astype(x.dtype)
        if j >= d_out:
            out_copy(j - d_out).wait()
        out_buf[j % d_out] = x * gate
        out_copy(j).start()
        if j + d_in < n_chunks:
            in_copy(j + d_in).start()
    for j in range(max(0, n_chunks - d_out), n_chunks):
        out_copy(j).wait()


def _block_diag(w, b):
    if b == 1:
        return w
    o, i = w.shape
    eye = jnp.eye(b, dtype=w.dtype)
    full = eye[:, :, None, None] * w[None, None, :, :]
    return full.transpose(0, 2, 1, 3).reshape(b * o, b * i)


def se_manual(x_nchw, w1, w2, S, ch_imgs, d_in, d_out):
    N, C, H, W = x_nchw.shape
    HW = H * W
    HWp = _ceil_to(HW, _LANE)
    dtype = x_nchw.dtype
    rows = ch_imgs * C
    n_chunks = N // ch_imgs

    x_flat = x_nchw.reshape(N, C, HW)
    if HWp != HW:
        x_flat = jnp.pad(x_flat, ((0, 0), (0, 0), (0, HWp - HW)))
    x2d = x_flat.reshape(N * C, HWp)

    w1d = _block_diag(w1, ch_imgs)
    w2d = _block_diag(w2, ch_imgs)

    chunk_bytes = rows * HWp * dtype.itemsize
    vmem_limit = int(min(60 << 20,
                         (d_in + d_out + 1) * chunk_bytes + (2 << 20)))

    out = pl.pallas_call(
        functools.partial(_se_manual_kernel, n_chunks=n_chunks, rows=rows,
                          d_in=d_in, d_out=d_out, inv_hw=1.0 / HW),
        out_shape=jax.ShapeDtypeStruct((N * C, HWp), dtype),
        in_specs=[
            pl.BlockSpec(memory_space=pl.ANY),
            pl.BlockSpec((ch_imgs * S, rows), lambda: (0, 0)),
            pl.BlockSpec((rows, ch_imgs * S), lambda: (0, 0)),
        ],
        out_specs=pl.BlockSpec(memory_space=pl.ANY),
        scratch_shapes=[
            pltpu.VMEM((d_in, rows, HWp), dtype),
            pltpu.VMEM((d_out, rows, HWp), dtype),
            pltpu.SemaphoreType.DMA((d_in,)),
            pltpu.SemaphoreType.DMA((d_out,)),
        ],
        compiler_params=pltpu.CompilerParams(
            vmem_limit_bytes=vmem_limit,
        ),
    )(x2d, w1d, w2d)

    out = out.reshape(N, C, HWp)
    if HWp != HW:
        out = out[:, :, :HW]
    return out.reshape(N, C, H, W)


def kernel(x_nchw, w_squeeze, w_unsqueeze):
    if w_squeeze.ndim == 4:
        w_squeeze = w_squeeze.reshape(w_squeeze.shape[0], w_squeeze.shape[1])
    if w_unsqueeze.ndim == 4:
        w_unsqueeze = w_unsqueeze.reshape(w_unsqueeze.shape[0],
                                          w_unsqueeze.shape[1])
    S = w_squeeze.shape[0]
    w1 = w_squeeze.astype(jnp.float32)
    w2 = w_unsqueeze.astype(jnp.float32)
    return se_manual(x_nchw, w1, w2, S, ch_imgs=2, d_in=5, d_out=2)
